# Initial kernel scaffold; baseline (speedup 1.0000x reference)
import jax
import jax.numpy as jnp
from jax.experimental import pallas as pl


def kernel(x, edge_feature, edge_index, in_table, out_table):
    raise NotImplementedError("write your pallas kernel here")


# speedup vs baseline: 6.7200x; 6.7200x over previous
import functools

import jax
import jax.numpy as jnp
from jax import lax
from jax.experimental import pallas as pl
from jax.experimental.pallas import tpu as pltpu
from jax.experimental.pallas import tpu_sc as plsc

NC = 2
NS = 16
NW = NC * NS
LANES = 16

CW = 80
GK = 25


def _wid():
    return lax.axis_index("s") * NC + lax.axis_index("c")


def _scatter_body(n, rpw, src_hbm, dst_hbm, s_hbm, srcb, dstb, keyb, eidb, sem):
    wid = _wid()
    row0 = wid * rpw
    pltpu.sync_copy(src_hbm.at[wid], srcb)
    pltpu.sync_copy(dst_hbm.at[wid], dstb)

    def compute_row(r, carry):
        for i in range(CW // LANES):
            sl = pl.ds(i * LANES, LANES)
            keyb[r, sl] = srcb[r, sl] * n + dstb[r, sl]
            eidb[r, sl] = (row0 + r) * CW + i * LANES + lax.iota(jnp.int32, 16)
        return carry

    lax.fori_loop(0, rpw, compute_row, 0)

    def group(g, carry):
        def fire(r, c):
            rr = g * GK + r
            pltpu.async_copy(eidb.at[rr], s_hbm.at[keyb.at[rr]], sem)
            return c

        lax.fori_loop(0, GK, fire, 0)

        def drain(r, c):
            rr = g * GK + r
            pltpu.make_async_copy(eidb.at[rr], s_hbm.at[keyb.at[rr]], sem).wait()
            return c

        lax.fori_loop(0, GK, drain, 0)
        return carry

    lax.fori_loop(0, rpw // GK, group, 0)


def _degree_body(n, rpw, src_hbm, dst_hbm, s_hbm, inp_hbm, outp_hbm,
                 srcb, dstb, keyb, eidb, gotb, valb, zb,
                 din_sh, dout_sh, sem):
    cid = lax.axis_index("c")
    sid = lax.axis_index("s")
    wid = _wid()
    row0 = wid * rpw

    @pl.when(sid == 0)
    def _zero():
        def zrow(i, c):
            zb[pl.ds(i * LANES, LANES)] = jnp.zeros((LANES,), jnp.int32)
            return c
        lax.fori_loop(0, n // LANES, zrow, 0)
        pltpu.sync_copy(zb, din_sh)
        pltpu.sync_copy(zb, dout_sh)

    pltpu.sync_copy(src_hbm.at[wid], srcb)
    pltpu.sync_copy(dst_hbm.at[wid], dstb)

    def compute_row(r, carry):
        for i in range(CW // LANES):
            sl = pl.ds(i * LANES, LANES)
            keyb[r, sl] = srcb[r, sl] * n + dstb[r, sl]
            eidb[r, sl] = (row0 + r) * CW + i * LANES + lax.iota(jnp.int32, 16)
        return carry

    lax.fori_loop(0, rpw, compute_row, 0)

    def ggroup(g, carry):
        def fire(r, c):
            rr = g * GK + r
            pltpu.async_copy(s_hbm.at[keyb.at[rr]], gotb.at[rr], sem)
            return c

        lax.fori_loop(0, GK, fire, 0)

        def drain(r, c):
            rr = g * GK + r
            pltpu.make_async_copy(s_hbm.at[keyb.at[rr]], gotb.at[rr], sem).wait()
            return c

        lax.fori_loop(0, GK, drain, 0)
        return carry

    lax.fori_loop(0, rpw // GK, ggroup, 0)

    def val_row(r, carry):
        for i in range(CW // LANES):
            sl = pl.ds(i * LANES, LANES)
            valb[r, sl] = 1 - jnp.minimum(jnp.abs(gotb[r, sl] - eidb[r, sl]), 1)
        return carry

    lax.fori_loop(0, rpw, val_row, 0)

    plsc.subcore_barrier()

    def sgroup(g, carry):
        def fire(r, c):
            rr = g * GK + r
            pltpu.async_copy(valb.at[rr], din_sh.at[srcb.at[rr]], sem, add=True)
            pltpu.async_copy(valb.at[rr], dout_sh.at[dstb.at[rr]], sem, add=True)
            return c

        lax.fori_loop(0, GK, fire, 0)

        def drain(r, c):
            rr = g * GK + r
            pltpu.make_async_copy(valb.at[rr], din_sh.at[srcb.at[rr]], sem).wait()
            pltpu.make_async_copy(valb.at[rr], dout_sh.at[dstb.at[rr]], sem).wait()
            return c

        lax.fori_loop(0, GK, drain, 0)
        return carry

    lax.fori_loop(0, rpw // GK, sgroup, 0)

    plsc.subcore_barrier()

    @pl.when(sid == 0)
    def _writeout():
        pltpu.sync_copy(din_sh, inp_hbm.at[cid, 0])
        pltpu.sync_copy(dout_sh, outp_hbm.at[cid, 0])


def _emb_body(num_emb, blk, ia0, ia1, oa0, oa1, x_ref, itab, otab, out_ref):
    din = jnp.minimum(ia0[0, 0, :] + ia1[0, 0, :], num_emb - 1)
    dout = jnp.minimum(oa0[0, 0, :] + oa1[0, 0, :], num_emb - 1)
    ioh = (lax.broadcasted_iota(jnp.int32, (blk, num_emb), 1)
           == din[:, None]).astype(jnp.float32)
    ooh = (lax.broadcasted_iota(jnp.int32, (blk, num_emb), 1)
           == dout[:, None]).astype(jnp.float32)
    g = jnp.dot(ioh, itab[...], preferred_element_type=jnp.float32)
    g = g + jnp.dot(ooh, otab[...], preferred_element_type=jnp.float32)
    out_ref[...] = x_ref[...] + g


def kernel(x, edge_feature, edge_index, in_table, out_table):
    n, d_node = x.shape
    e = edge_index.shape[1]
    num_emb = in_table.shape[0]

    rows = e // CW
    rpw = rows // NW

    src3d = edge_index[0].reshape(NW, rpw, CW)
    dst3d = edge_index[1].reshape(NW, rpw, CW)

    mesh = plsc.VectorSubcoreMesh(core_axis_name="c", subcore_axis_name="s")

    scatter_k = functools.partial(
        pl.kernel,
        out_type=jax.ShapeDtypeStruct((n * n,), jnp.int32),
        mesh=mesh,
        scratch_types=[
            pltpu.VMEM((rpw, CW), jnp.int32),
            pltpu.VMEM((rpw, CW), jnp.int32),
            pltpu.VMEM((rpw, CW), jnp.int32),
            pltpu.VMEM((rpw, CW), jnp.int32),
            pltpu.SemaphoreType.DMA,
        ],
    )(functools.partial(_scatter_body, n, rpw))

    slot = scatter_k(src3d, dst3d)

    degree_k = functools.partial(
        pl.kernel,
        out_type=(
            jax.ShapeDtypeStruct((NC, 1, n), jnp.int32),
            jax.ShapeDtypeStruct((NC, 1, n), jnp.int32),
        ),
        mesh=mesh,
        scratch_types=[
            pltpu.VMEM((rpw, CW), jnp.int32),
            pltpu.VMEM((rpw, CW), jnp.int32),
            pltpu.VMEM((rpw, CW), jnp.int32),
            pltpu.VMEM((rpw, CW), jnp.int32),
            pltpu.VMEM((rpw, CW), jnp.int32),
            pltpu.VMEM((rpw, CW), jnp.int32),
            pltpu.VMEM((n,), jnp.int32),
            pltpu.VMEM_SHARED((n,), jnp.int32),
            pltpu.VMEM_SHARED((n,), jnp.int32),
            pltpu.SemaphoreType.DMA,
        ],
    )(functools.partial(_degree_body, n, rpw))

    in_part, out_part = degree_k(src3d, dst3d, slot)

    blk = 1000
    nblk = n // blk
    ia0 = in_part[0, 0].reshape(nblk, 1, blk)
    ia1 = in_part[1, 0].reshape(nblk, 1, blk)
    oa0 = out_part[0, 0].reshape(nblk, 1, blk)
    oa1 = out_part[1, 0].reshape(nblk, 1, blk)

    part_spec = pl.BlockSpec((1, 1, blk), lambda j: (j, 0, 0))
    tab_spec = pl.BlockSpec((num_emb, d_node), lambda j: (0, 0))
    row_spec = pl.BlockSpec((blk, d_node), lambda j: (j, 0))

    node_feature = pl.pallas_call(
        functools.partial(_emb_body, num_emb, blk),
        grid=(nblk,),
        in_specs=[part_spec, part_spec, part_spec, part_spec,
                  row_spec, tab_spec, tab_spec],
        out_specs=row_spec,
        out_shape=jax.ShapeDtypeStruct((n, d_node), jnp.float32),
    )(ia0, ia1, oa0, oa1, x, in_table, out_table)

    return (node_feature, 0)


# speedup vs baseline: 6.7218x; 1.0003x over previous
import functools

import jax
import jax.numpy as jnp
from jax import lax
from jax.experimental import pallas as pl
from jax.experimental.pallas import tpu as pltpu
from jax.experimental.pallas import tpu_sc as plsc

NC = 2
NS = 16
NW = NC * NS
LANES = 16

CW = 80
GK = 25


def _wid():
    return lax.axis_index("s") * NC + lax.axis_index("c")


def _scatter_body(n, rpw, src_hbm, dst_hbm, s_hbm, srcb, dstb, keyb, eidb, sem):
    wid = _wid()
    row0 = wid * rpw
    pltpu.sync_copy(src_hbm.at[wid], srcb)
    pltpu.sync_copy(dst_hbm.at[wid], dstb)

    def compute_row(r, carry):
        for i in range(CW // LANES):
            sl = pl.ds(i * LANES, LANES)
            keyb[r, sl] = srcb[r, sl] * n + dstb[r, sl]
            eidb[r, sl] = (row0 + r) * CW + i * LANES + lax.iota(jnp.int32, 16)
        return carry

    lax.fori_loop(0, rpw, compute_row, 0)

    def group(g, carry):
        def fire(r, c):
            rr = g * GK + r
            pltpu.async_copy(eidb.at[rr], s_hbm.at[keyb.at[rr]], sem)
            return c

        lax.fori_loop(0, GK, fire, 0)

        def drain(r, c):
            rr = g * GK + r
            pltpu.make_async_copy(eidb.at[rr], s_hbm.at[keyb.at[rr]], sem).wait()
            return c

        lax.fori_loop(0, GK, drain, 0)
        return carry

    lax.fori_loop(0, rpw // GK, group, 0)


def _degree_body(n, rpw, src_hbm, dst_hbm, s_hbm, inp_hbm, outp_hbm,
                 srcb, dstb, keyb, eidb, gotb, valb, zb,
                 din_sh, dout_sh, sem):
    cid = lax.axis_index("c")
    sid = lax.axis_index("s")
    wid = _wid()
    row0 = wid * rpw

    @pl.when(sid == 0)
    def _zero():
        def zrow(i, c):
            zb[pl.ds(i * LANES, LANES)] = jnp.zeros((LANES,), jnp.int32)
            return c
        lax.fori_loop(0, n // LANES, zrow, 0)
        pltpu.sync_copy(zb, din_sh)
        pltpu.sync_copy(zb, dout_sh)

    pltpu.sync_copy(src_hbm.at[wid], srcb)
    pltpu.sync_copy(dst_hbm.at[wid], dstb)

    def compute_row(r, carry):
        for i in range(CW // LANES):
            sl = pl.ds(i * LANES, LANES)
            keyb[r, sl] = srcb[r, sl] * n + dstb[r, sl]
            eidb[r, sl] = (row0 + r) * CW + i * LANES + lax.iota(jnp.int32, 16)
        return carry

    lax.fori_loop(0, rpw, compute_row, 0)

    def ggroup(g, carry):
        def fire(r, c):
            rr = g * GK + r
            pltpu.async_copy(s_hbm.at[keyb.at[rr]], gotb.at[rr], sem)
            return c

        lax.fori_loop(0, GK, fire, 0)

        def drain(r, c):
            rr = g * GK + r
            pltpu.make_async_copy(s_hbm.at[keyb.at[rr]], gotb.at[rr], sem).wait()
            return c

        lax.fori_loop(0, GK, drain, 0)
        return carry

    lax.fori_loop(0, rpw // GK, ggroup, 0)

    def val_row(r, carry):
        for i in range(CW // LANES):
            sl = pl.ds(i * LANES, LANES)
            valb[r, sl] = 1 - jnp.minimum(jnp.abs(gotb[r, sl] - eidb[r, sl]), 1)
        return carry

    lax.fori_loop(0, rpw, val_row, 0)

    plsc.subcore_barrier()

    def sgroup(g, carry):
        def fire(r, c):
            rr = g * GK + r
            pltpu.async_copy(valb.at[rr], din_sh.at[srcb.at[rr]], sem, add=True)
            pltpu.async_copy(valb.at[rr], dout_sh.at[dstb.at[rr]], sem, add=True)
            return c

        lax.fori_loop(0, GK, fire, 0)

        def drain(r, c):
            rr = g * GK + r
            pltpu.make_async_copy(valb.at[rr], din_sh.at[srcb.at[rr]], sem).wait()
            pltpu.make_async_copy(valb.at[rr], dout_sh.at[dstb.at[rr]], sem).wait()
            return c

        lax.fori_loop(0, GK, drain, 0)
        return carry

    lax.fori_loop(0, rpw // GK, sgroup, 0)

    plsc.subcore_barrier()

    @pl.when(sid == 0)
    def _writeout():
        pltpu.sync_copy(din_sh, inp_hbm.at[cid, 0])
        pltpu.sync_copy(dout_sh, outp_hbm.at[cid, 0])


def _emb_body(num_emb, blk, ia0, ia1, oa0, oa1, x_ref, itab, otab, out_ref):
    din = jnp.minimum(ia0[0, 0, :] + ia1[0, 0, :], num_emb - 1)
    dout = jnp.minimum(oa0[0, 0, :] + oa1[0, 0, :], num_emb - 1)
    ioh = (lax.broadcasted_iota(jnp.int32, (blk, num_emb), 1)
           == din[:, None]).astype(jnp.float32)
    ooh = (lax.broadcasted_iota(jnp.int32, (blk, num_emb), 1)
           == dout[:, None]).astype(jnp.float32)
    g = jnp.dot(ioh, itab[...], preferred_element_type=jnp.float32)
    g = g + jnp.dot(ooh, otab[...], preferred_element_type=jnp.float32)
    out_ref[...] = x_ref[...] + g


def kernel(x, edge_feature, edge_index, in_table, out_table):
    n, d_node = x.shape
    e = edge_index.shape[1]
    num_emb = in_table.shape[0]

    rows = e // CW
    rpw = rows // NW

    src3d = edge_index[0].reshape(NW, rpw, CW)
    dst3d = edge_index[1].reshape(NW, rpw, CW)

    mesh = plsc.VectorSubcoreMesh(core_axis_name="c", subcore_axis_name="s")

    scatter_k = functools.partial(
        pl.kernel,
        out_type=jax.ShapeDtypeStruct((n * n,), jnp.int32),
        mesh=mesh,
        scratch_types=[
            pltpu.VMEM((rpw, CW), jnp.int32),
            pltpu.VMEM((rpw, CW), jnp.int32),
            pltpu.VMEM((rpw, CW), jnp.int32),
            pltpu.VMEM((rpw, CW), jnp.int32),
            pltpu.SemaphoreType.DMA,
        ],
        name="p1_scatter",
    )(functools.partial(_scatter_body, n, rpw))

    slot = scatter_k(src3d, dst3d)

    degree_k = functools.partial(
        pl.kernel,
        out_type=(
            jax.ShapeDtypeStruct((NC, 1, n), jnp.int32),
            jax.ShapeDtypeStruct((NC, 1, n), jnp.int32),
        ),
        mesh=mesh,
        scratch_types=[
            pltpu.VMEM((rpw, CW), jnp.int32),
            pltpu.VMEM((rpw, CW), jnp.int32),
            pltpu.VMEM((rpw, CW), jnp.int32),
            pltpu.VMEM((rpw, CW), jnp.int32),
            pltpu.VMEM((rpw, CW), jnp.int32),
            pltpu.VMEM((rpw, CW), jnp.int32),
            pltpu.VMEM((n,), jnp.int32),
            pltpu.VMEM_SHARED((n,), jnp.int32),
            pltpu.VMEM_SHARED((n,), jnp.int32),
            pltpu.SemaphoreType.DMA,
        ],
        name="p2_degree",
    )(functools.partial(_degree_body, n, rpw))

    in_part, out_part = degree_k(src3d, dst3d, slot)

    blk = 1000
    nblk = n // blk
    ia0 = in_part[0, 0].reshape(nblk, 1, blk)
    ia1 = in_part[1, 0].reshape(nblk, 1, blk)
    oa0 = out_part[0, 0].reshape(nblk, 1, blk)
    oa1 = out_part[1, 0].reshape(nblk, 1, blk)

    part_spec = pl.BlockSpec((1, 1, blk), lambda j: (j, 0, 0))
    tab_spec = pl.BlockSpec((num_emb, d_node), lambda j: (0, 0))
    row_spec = pl.BlockSpec((blk, d_node), lambda j: (j, 0))

    node_feature = pl.pallas_call(
        functools.partial(_emb_body, num_emb, blk),
        grid=(nblk,),
        in_specs=[part_spec, part_spec, part_spec, part_spec,
                  row_spec, tab_spec, tab_spec],
        out_specs=row_spec,
        out_shape=jax.ShapeDtypeStruct((n, d_node), jnp.float32),
    )(ia0, ia1, oa0, oa1, x, in_table, out_table)

    return (node_feature, 0)


# speedup vs baseline: 6.7689x; 1.0070x over previous
import functools

import jax
import jax.numpy as jnp
from jax import lax
from jax.experimental import pallas as pl
from jax.experimental.pallas import tpu as pltpu
from jax.experimental.pallas import tpu_sc as plsc

NC = 2
NS = 16
NW = NC * NS
LANES = 16

CW = 80
GK = 125


def _wid():
    return lax.axis_index("s") * NC + lax.axis_index("c")


def _scatter_body(n, rpw, src_hbm, dst_hbm, s_hbm, srcb, dstb, keyb, eidb, sem):
    wid = _wid()
    row0 = wid * rpw
    pltpu.sync_copy(src_hbm.at[wid], srcb)
    pltpu.sync_copy(dst_hbm.at[wid], dstb)

    def compute_row(r, carry):
        for i in range(CW // LANES):
            sl = pl.ds(i * LANES, LANES)
            keyb[r, sl] = srcb[r, sl] * n + dstb[r, sl]
            eidb[r, sl] = (row0 + r) * CW + i * LANES + lax.iota(jnp.int32, 16)
        return carry

    lax.fori_loop(0, rpw, compute_row, 0)

    def group(g, carry):
        def fire(r, c):
            rr = g * GK + r
            pltpu.async_copy(eidb.at[rr], s_hbm.at[keyb.at[rr]], sem)
            return c

        lax.fori_loop(0, GK, fire, 0)

        def drain(r, c):
            rr = g * GK + r
            pltpu.make_async_copy(eidb.at[rr], s_hbm.at[keyb.at[rr]], sem).wait()
            return c

        lax.fori_loop(0, GK, drain, 0)
        return carry

    lax.fori_loop(0, rpw // GK, group, 0)


def _degree_body(n, rpw, src_hbm, dst_hbm, s_hbm, inp_hbm, outp_hbm,
                 srcb, dstb, keyb, eidb, gotb, valb, zb,
                 din_sh, dout_sh, sem):
    cid = lax.axis_index("c")
    sid = lax.axis_index("s")
    wid = _wid()
    row0 = wid * rpw

    @pl.when(sid == 0)
    def _zero():
        def zrow(i, c):
            zb[pl.ds(i * LANES, LANES)] = jnp.zeros((LANES,), jnp.int32)
            return c
        lax.fori_loop(0, n // LANES, zrow, 0)
        pltpu.sync_copy(zb, din_sh)
        pltpu.sync_copy(zb, dout_sh)

    pltpu.sync_copy(src_hbm.at[wid], srcb)
    pltpu.sync_copy(dst_hbm.at[wid], dstb)

    def compute_row(r, carry):
        for i in range(CW // LANES):
            sl = pl.ds(i * LANES, LANES)
            keyb[r, sl] = srcb[r, sl] * n + dstb[r, sl]
            eidb[r, sl] = (row0 + r) * CW + i * LANES + lax.iota(jnp.int32, 16)
        return carry

    lax.fori_loop(0, rpw, compute_row, 0)

    def ggroup(g, carry):
        def fire(r, c):
            rr = g * GK + r
            pltpu.async_copy(s_hbm.at[keyb.at[rr]], gotb.at[rr], sem)
            return c

        lax.fori_loop(0, GK, fire, 0)

        def drain(r, c):
            rr = g * GK + r
            pltpu.make_async_copy(s_hbm.at[keyb.at[rr]], gotb.at[rr], sem).wait()
            return c

        lax.fori_loop(0, GK, drain, 0)
        return carry

    lax.fori_loop(0, rpw // GK, ggroup, 0)

    def val_row(r, carry):
        for i in range(CW // LANES):
            sl = pl.ds(i * LANES, LANES)
            valb[r, sl] = 1 - jnp.minimum(jnp.abs(gotb[r, sl] - eidb[r, sl]), 1)
        return carry

    lax.fori_loop(0, rpw, val_row, 0)

    plsc.subcore_barrier()

    def sgroup(g, carry):
        def fire(r, c):
            rr = g * GK + r
            pltpu.async_copy(valb.at[rr], din_sh.at[srcb.at[rr]], sem, add=True)
            pltpu.async_copy(valb.at[rr], dout_sh.at[dstb.at[rr]], sem, add=True)
            return c

        lax.fori_loop(0, GK, fire, 0)

        def drain(r, c):
            rr = g * GK + r
            pltpu.make_async_copy(valb.at[rr], din_sh.at[srcb.at[rr]], sem).wait()
            pltpu.make_async_copy(valb.at[rr], dout_sh.at[dstb.at[rr]], sem).wait()
            return c

        lax.fori_loop(0, GK, drain, 0)
        return carry

    lax.fori_loop(0, rpw // GK, sgroup, 0)

    plsc.subcore_barrier()

    @pl.when(sid == 0)
    def _writeout():
        pltpu.sync_copy(din_sh, inp_hbm.at[cid, 0])
        pltpu.sync_copy(dout_sh, outp_hbm.at[cid, 0])


def _emb_body(num_emb, blk, ia0, ia1, oa0, oa1, x_ref, itab, otab, out_ref):
    din = jnp.minimum(ia0[0, 0, :] + ia1[0, 0, :], num_emb - 1)
    dout = jnp.minimum(oa0[0, 0, :] + oa1[0, 0, :], num_emb - 1)
    ioh = (lax.broadcasted_iota(jnp.int32, (blk, num_emb), 1)
           == din[:, None]).astype(jnp.float32)
    ooh = (lax.broadcasted_iota(jnp.int32, (blk, num_emb), 1)
           == dout[:, None]).astype(jnp.float32)
    g = jnp.dot(ioh, itab[...], preferred_element_type=jnp.float32)
    g = g + jnp.dot(ooh, otab[...], preferred_element_type=jnp.float32)
    out_ref[...] = x_ref[...] + g


def kernel(x, edge_feature, edge_index, in_table, out_table):
    n, d_node = x.shape
    e = edge_index.shape[1]
    num_emb = in_table.shape[0]

    rows = e // CW
    rpw = rows // NW

    src3d = edge_index[0].reshape(NW, rpw, CW)
    dst3d = edge_index[1].reshape(NW, rpw, CW)

    mesh = plsc.VectorSubcoreMesh(core_axis_name="c", subcore_axis_name="s")

    scatter_k = functools.partial(
        pl.kernel,
        out_type=jax.ShapeDtypeStruct((n * n,), jnp.int32),
        mesh=mesh,
        scratch_types=[
            pltpu.VMEM((rpw, CW), jnp.int32),
            pltpu.VMEM((rpw, CW), jnp.int32),
            pltpu.VMEM((rpw, CW), jnp.int32),
            pltpu.VMEM((rpw, CW), jnp.int32),
            pltpu.SemaphoreType.DMA,
        ],
        name="p1_scatter",
    )(functools.partial(_scatter_body, n, rpw))

    slot = scatter_k(src3d, dst3d)

    degree_k = functools.partial(
        pl.kernel,
        out_type=(
            jax.ShapeDtypeStruct((NC, 1, n), jnp.int32),
            jax.ShapeDtypeStruct((NC, 1, n), jnp.int32),
        ),
        mesh=mesh,
        scratch_types=[
            pltpu.VMEM((rpw, CW), jnp.int32),
            pltpu.VMEM((rpw, CW), jnp.int32),
            pltpu.VMEM((rpw, CW), jnp.int32),
            pltpu.VMEM((rpw, CW), jnp.int32),
            pltpu.VMEM((rpw, CW), jnp.int32),
            pltpu.VMEM((rpw, CW), jnp.int32),
            pltpu.VMEM((n,), jnp.int32),
            pltpu.VMEM_SHARED((n,), jnp.int32),
            pltpu.VMEM_SHARED((n,), jnp.int32),
            pltpu.SemaphoreType.DMA,
        ],
        name="p2_degree",
    )(functools.partial(_degree_body, n, rpw))

    in_part, out_part = degree_k(src3d, dst3d, slot)

    blk = 1000
    nblk = n // blk
    ia0 = in_part[0, 0].reshape(nblk, 1, blk)
    ia1 = in_part[1, 0].reshape(nblk, 1, blk)
    oa0 = out_part[0, 0].reshape(nblk, 1, blk)
    oa1 = out_part[1, 0].reshape(nblk, 1, blk)

    part_spec = pl.BlockSpec((1, 1, blk), lambda j: (j, 0, 0))
    tab_spec = pl.BlockSpec((num_emb, d_node), lambda j: (0, 0))
    row_spec = pl.BlockSpec((blk, d_node), lambda j: (j, 0))

    node_feature = pl.pallas_call(
        functools.partial(_emb_body, num_emb, blk),
        grid=(nblk,),
        in_specs=[part_spec, part_spec, part_spec, part_spec,
                  row_spec, tab_spec, tab_spec],
        out_specs=row_spec,
        out_shape=jax.ShapeDtypeStruct((n, d_node), jnp.float32),
    )(ia0, ia1, oa0, oa1, x, in_table, out_table)

    return (node_feature, 0)
